# baseline (speedup 1.0000x reference)
import functools

import jax
import jax.numpy as jnp
from jax import lax
from jax.experimental import pallas as pl
from jax.experimental.pallas import tpu as pltpu
from jax.experimental.pallas import tpu_sc as plsc

B, S, D = 16, 512, 1024
NC, NS, L = 2, 16, 16
NW = NC * NS
ROWS = B * S
HB = B // 2
HROWS = HB * S
RPW = HROWS // NW
WPB = S // RPW
CHUNK = 32
NCHUNK = RPW // CHUNK
NBUF = 3
BS = 64
GT = S // BS


def _make_sc():
    mesh = plsc.VectorSubcoreMesh(core_axis_name="c", subcore_axis_name="s")

    @functools.partial(
        pl.kernel,
        mesh=mesh,
        out_type=jax.ShapeDtypeStruct((HROWS, D), jnp.float32),
        scratch_types=(
            [pltpu.VMEM((RPW,), jnp.int32)]
            + [pltpu.VMEM((CHUNK, D), jnp.float32) for _ in range(NBUF)]
            + [pltpu.SemaphoreType.DMA for _ in range(2 * NBUF)]
        ),
    )
    def k(x_hbm, perm_hbm, out_hbm, idx_v, b0, b1, b2, g0, g1, g2, s0, s1, s2):
        wid = lax.axis_index("s") * NC + lax.axis_index("c")
        b = wid // WPB
        jbase = (wid % WPB) * RPW
        row_off = b * S
        obase = row_off + jbase

        pltpu.sync_copy(perm_hbm.at[pl.ds(jbase, RPW)], idx_v)
        for i in range(RPW // L):
            sl = pl.ds(i * L, L)
            idx_v[sl] = idx_v[sl] + row_off

        bufs = (b0, b1, b2)
        gsem = (g0, g1, g2)
        ssem = (s0, s1, s2)
        store = [None] * NCHUNK
        for c in range(NCHUNK):
            p = c % NBUF
            if c >= NBUF:
                store[c - NBUF].wait()
            pltpu.async_copy(
                x_hbm.at[idx_v.at[pl.ds(c * CHUNK, CHUNK)]], bufs[p],
                gsem[p]).wait()
            store[c] = pltpu.async_copy(
                bufs[p], out_hbm.at[pl.ds(obase + c * CHUNK, CHUNK)], ssem[p])
        for c in range(NCHUNK - NBUF, NCHUNK):
            if store[c] is not None:
                store[c].wait()

    return k


_sc_half = _make_sc()


def _tc_body(x_ref, o_ref):
    for i in range(BS):
        o_ref[:, i, :] = x_ref[:, BS - 1 - i, :]


def _tc_half(x):
    return pl.pallas_call(
        _tc_body,
        grid=(GT,),
        in_specs=[pl.BlockSpec((HB, BS, D), lambda g: (1, GT - 1 - g, 0))],
        out_specs=pl.BlockSpec((HB, BS, D), lambda g: (0, g, 0)),
        out_shape=jax.ShapeDtypeStruct((HB, S, D), jnp.float32),
    )(x)


def kernel(x, perm):
    x_flat = x.reshape(ROWS, D)
    sc_out = _sc_half(x_flat, perm).reshape(HB, S, D)
    tc_out = _tc_half(x)
    return jnp.concatenate([sc_out, tc_out], axis=0)


# speedup vs baseline: 1.3228x; 1.3228x over previous
import functools

import jax
import jax.numpy as jnp
from jax import lax
from jax.experimental import pallas as pl
from jax.experimental.pallas import tpu as pltpu
from jax.experimental.pallas import tpu_sc as plsc

B, S, D = 16, 512, 1024
NC, NS, L = 2, 16, 16
NW = NC * NS
ROWS = B * S
HB = B // 2
HROWS = HB * S
RPW = HROWS // NW
WPB = S // RPW
CHUNK = 32
NCHUNK = RPW // CHUNK
NBUF = 3
BS = 64
GT = S // BS


def _make_sc():
    mesh = plsc.VectorSubcoreMesh(core_axis_name="c", subcore_axis_name="s")

    @functools.partial(
        pl.kernel,
        mesh=mesh,
        out_type=jax.ShapeDtypeStruct((ROWS, D), jnp.float32),
        scratch_types=(
            [pltpu.VMEM((RPW,), jnp.int32)]
            + [pltpu.VMEM((CHUNK, D), jnp.float32) for _ in range(NBUF)]
            + [pltpu.SemaphoreType.DMA for _ in range(2 * NBUF)]
        ),
    )
    def k(x_hbm, perm_hbm, out_hbm, idx_v, b0, b1, b2, g0, g1, g2, s0, s1, s2):
        wid = lax.axis_index("s") * NC + lax.axis_index("c")
        b = wid // WPB
        jbase = (wid % WPB) * RPW
        row_off = b * S
        obase = row_off + jbase

        pltpu.sync_copy(perm_hbm.at[pl.ds(jbase, RPW)], idx_v)
        for i in range(RPW // L):
            sl = pl.ds(i * L, L)
            idx_v[sl] = idx_v[sl] + row_off

        bufs = (b0, b1, b2)
        gsem = (g0, g1, g2)
        ssem = (s0, s1, s2)
        store = [None] * NCHUNK
        for c in range(NCHUNK):
            p = c % NBUF
            if c >= NBUF:
                store[c - NBUF].wait()
            pltpu.async_copy(
                x_hbm.at[idx_v.at[pl.ds(c * CHUNK, CHUNK)]], bufs[p],
                gsem[p]).wait()
            store[c] = pltpu.async_copy(
                bufs[p], out_hbm.at[pl.ds(obase + c * CHUNK, CHUNK)], ssem[p])
        for c in range(NCHUNK - NBUF, NCHUNK):
            if store[c] is not None:
                store[c].wait()

    return k


_sc_half = _make_sc()


def _tc_body(x_ref, acc_ref, o_ref):
    del acc_ref
    for i in range(BS):
        o_ref[:, i, :] = x_ref[:, BS - 1 - i, :]


def _tc_finish(x, acc):
    return pl.pallas_call(
        _tc_body,
        grid=(GT,),
        in_specs=[
            pl.BlockSpec((HB, BS, D), lambda g: (1, GT - 1 - g, 0)),
            pl.BlockSpec(memory_space=pl.ANY),
        ],
        out_specs=pl.BlockSpec((HB, BS, D), lambda g: (1, g, 0)),
        out_shape=jax.ShapeDtypeStruct((B, S, D), jnp.float32),
        input_output_aliases={1: 0},
    )(x, acc)


def kernel(x, perm):
    x_flat = x.reshape(ROWS, D)
    sc_full = _sc_half(x_flat, perm).reshape(B, S, D)
    return _tc_finish(x, sc_full)


# speedup vs baseline: 1.3843x; 1.0465x over previous
import functools

import jax
import jax.numpy as jnp
from jax import lax
from jax.experimental import pallas as pl
from jax.experimental.pallas import tpu as pltpu
from jax.experimental.pallas import tpu_sc as plsc

B, S, D = 16, 512, 1024
NC, NS, L = 2, 16, 16
NW = NC * NS
ROWS = B * S
RPW = ROWS // NW
CHUNK = 32
NCHUNK = RPW // CHUNK
NBUF = 3


def _make_kernel():
    mesh = plsc.VectorSubcoreMesh(core_axis_name="c", subcore_axis_name="s")

    @functools.partial(
        pl.kernel,
        mesh=mesh,
        out_type=jax.ShapeDtypeStruct((ROWS, D), jnp.float32),
        scratch_types=(
            [pltpu.VMEM((RPW,), jnp.int32)]
            + [pltpu.VMEM((CHUNK, D), jnp.float32) for _ in range(NBUF)]
            + [pltpu.SemaphoreType.DMA for _ in range(2 * NBUF)]
        ),
    )
    def k(x_hbm, perm_hbm, out_hbm, idx_v, b0, b1, b2, g0, g1, g2, s0, s1, s2):
        wid = lax.axis_index("s") * NC + lax.axis_index("c")
        b = wid // 2
        jbase = (wid % 2) * RPW
        row_off = b * S
        obase = row_off + jbase

        pltpu.sync_copy(perm_hbm.at[pl.ds(jbase, RPW)], idx_v)
        for i in range(RPW // L):
            sl = pl.ds(i * L, L)
            idx_v[sl] = idx_v[sl] + row_off

        bufs = (b0, b1, b2)
        gsem = (g0, g1, g2)
        ssem = (s0, s1, s2)
        store = [None] * NCHUNK
        for c in range(NCHUNK):
            p = c % NBUF
            if c >= NBUF:
                store[c - NBUF].wait()
            pltpu.async_copy(
                x_hbm.at[idx_v.at[pl.ds(c * CHUNK, CHUNK)]], bufs[p],
                gsem[p]).wait()
            store[c] = pltpu.async_copy(
                bufs[p], out_hbm.at[pl.ds(obase + c * CHUNK, CHUNK)], ssem[p])
        for c in range(NCHUNK - NBUF, NCHUNK):
            store[c].wait()

    return k


_sc_gather = _make_kernel()


def kernel(x, perm):
    x_flat = x.reshape(ROWS, D)
    out_flat = _sc_gather(x_flat, perm)
    return out_flat.reshape(B, S, D)
